# baseline (speedup 1.0000x reference)
import functools

import jax
import jax.numpy as jnp
from jax import lax
from jax.experimental import pallas as pl
from jax.experimental.pallas import tpu as pltpu
from jax.experimental.pallas import tpu_sc as plsc

_TILE = 2048
_CHUNK = 512


def _make_sc_gather(b, d):
    info = plsc.get_sparse_core_info()
    nc, ns = info.num_cores, info.num_subcores
    nw = nc * ns
    bpw = b // nw
    mesh = plsc.VectorSubcoreMesh(core_axis_name="c", subcore_axis_name="s")

    @functools.partial(
        pl.kernel,
        mesh=mesh,
        out_type=jax.ShapeDtypeStruct((b, d), jnp.float32),
        scratch_types=[
            pltpu.VMEM((bpw,), jnp.int32),
            pltpu.VMEM((bpw, d), jnp.float32),
            pltpu.SemaphoreType.DMA,
        ],
    )
    def gather_kernel(table_hbm, idx_hbm, out_hbm, idx_v, rows_v, sem):
        wid = lax.axis_index("s") * nc + lax.axis_index("c")
        base = wid * bpw
        pltpu.sync_copy(idx_hbm.at[pl.ds(base, bpw)], idx_v)
        pltpu.async_copy(table_hbm.at[idx_v], rows_v, sem).wait()
        pltpu.sync_copy(rows_v, out_hbm.at[pl.ds(base, bpw)])

    return gather_kernel


def _logz_body(qt_ref, e_ref, out_ref, m_ref, s_ref, *, n):
    i = pl.program_id(0)
    nsteps = pl.num_programs(0)

    @pl.when(i == 0)
    def _():
        m_ref[...] = jnp.full(m_ref.shape, -jnp.inf, m_ref.dtype)
        s_ref[...] = jnp.zeros(s_ref.shape, s_ref.dtype)

    def update(scores):
        m_old = m_ref[...]
        bm = jnp.max(scores, axis=0, keepdims=True).astype(jnp.float32)
        m_new = jnp.maximum(m_old, bm)
        t = jnp.sum(
            jnp.exp(scores - m_new.astype(jnp.bfloat16)), axis=0, keepdims=True
        ).astype(jnp.float32)
        s_ref[...] = s_ref[...] * jnp.exp(m_old - m_new) + t
        m_ref[...] = m_new

    def chunk_scores(c):
        return lax.dot_general(
            e_ref[pl.ds(c * _CHUNK, _CHUNK), :].astype(jnp.bfloat16),
            qt_ref[...],
            (((1,), (0,)), ((), ())),
            preferred_element_type=jnp.float32,
        ).astype(jnp.bfloat16)

    @pl.when(i < nsteps - 1)
    def _():
        for c in range(_TILE // _CHUNK):
            update(chunk_scores(c))

    @pl.when(i == nsteps - 1)
    def _():
        for c in range(_TILE // _CHUNK):
            row = (
                lax.broadcasted_iota(jnp.int32, (_CHUNK, 1), 0)
                + i * _TILE
                + c * _CHUNK
            )
            update(jnp.where(row < n, chunk_scores(c), -jnp.inf))
        out_ref[...] = m_ref[...] + jnp.log(s_ref[...])


def _combine_body(q_ref, g_ref, logz_ref, out_ref):
    b = q_ref.shape[0]
    loss = (jnp.sum(logz_ref[...]) - jnp.sum(q_ref[...] * g_ref[...])) / b
    out_ref[...] = jnp.full((1, 1), loss, out_ref.dtype)


def kernel(query_embs, ent_embs, triplets):
    b, h = query_embs.shape
    n = ent_embs.shape[0]
    labels = triplets[:, 2].astype(jnp.int32)

    g = _make_sc_gather(b, h)(ent_embs, labels)

    grid = pl.cdiv(n, _TILE)
    logz = pl.pallas_call(
        functools.partial(_logz_body, n=n),
        grid=(grid,),
        in_specs=[
            pl.BlockSpec((h, b), lambda i: (0, 0)),
            pl.BlockSpec((_TILE, h), lambda i: (i, 0)),
        ],
        out_specs=pl.BlockSpec((1, b), lambda i: (0, 0)),
        out_shape=jax.ShapeDtypeStruct((1, b), jnp.float32),
        scratch_shapes=[
            pltpu.VMEM((1, b), jnp.float32),
            pltpu.VMEM((1, b), jnp.float32),
        ],
    )(query_embs.T.astype(jnp.bfloat16), ent_embs)

    loss = pl.pallas_call(
        _combine_body,
        out_shape=jax.ShapeDtypeStruct((1, 1), jnp.float32),
    )(query_embs, g, logz)
    return loss[0, 0]


# speedup vs baseline: 1.2746x; 1.2746x over previous
import functools

import jax
import jax.numpy as jnp
from jax import lax
from jax.experimental import pallas as pl
from jax.experimental.pallas import tpu as pltpu
from jax.experimental.pallas import tpu_sc as plsc

_TILE = 2048
_CHUNK = 512


def _make_sc_gather(b, d):
    info = plsc.get_sparse_core_info()
    nc, ns = info.num_cores, info.num_subcores
    nw = nc * ns
    bpw = b // nw
    mesh = plsc.VectorSubcoreMesh(core_axis_name="c", subcore_axis_name="s")

    @functools.partial(
        pl.kernel,
        mesh=mesh,
        out_type=jax.ShapeDtypeStruct((b, d), jnp.float32),
        scratch_types=[
            pltpu.VMEM((bpw,), jnp.int32),
            pltpu.VMEM((bpw, d), jnp.float32),
            pltpu.SemaphoreType.DMA,
        ],
    )
    def gather_kernel(table_hbm, idx_hbm, out_hbm, idx_v, rows_v, sem):
        wid = lax.axis_index("s") * nc + lax.axis_index("c")
        base = wid * bpw
        pltpu.sync_copy(idx_hbm.at[pl.ds(base, bpw)], idx_v)
        pltpu.async_copy(table_hbm.at[idx_v], rows_v, sem).wait()
        pltpu.sync_copy(rows_v, out_hbm.at[pl.ds(base, bpw)])

    return gather_kernel


def _logz_body(qt_ref, e_ref, out_ref, m_ref, s_ref, *, n):
    i = pl.program_id(0)
    nsteps = pl.num_programs(0)

    @pl.when(i == 0)
    def _():
        m_ref[...] = jnp.full(m_ref.shape, -jnp.inf, m_ref.dtype)
        s_ref[...] = jnp.zeros(s_ref.shape, s_ref.dtype)

    def update(scores):
        m_old = m_ref[...]
        bm = jnp.max(scores, axis=0, keepdims=True).astype(jnp.float32)
        m_new = jnp.maximum(m_old, bm)
        t = jnp.sum(
            jnp.exp(scores - m_new.astype(jnp.bfloat16)),
            axis=0,
            keepdims=True,
            dtype=jnp.bfloat16,
        ).astype(jnp.float32)
        s_ref[...] = s_ref[...] * jnp.exp(m_old - m_new) + t
        m_ref[...] = m_new

    def chunk_scores(c):
        return lax.dot_general(
            e_ref[pl.ds(c * _CHUNK, _CHUNK), :].astype(jnp.bfloat16),
            qt_ref[...],
            (((1,), (0,)), ((), ())),
            preferred_element_type=jnp.float32,
        ).astype(jnp.bfloat16)

    @pl.when(i < nsteps - 1)
    def _():
        for c in range(_TILE // _CHUNK):
            update(chunk_scores(c))

    @pl.when(i == nsteps - 1)
    def _():
        for c in range(_TILE // _CHUNK):
            row = (
                lax.broadcasted_iota(jnp.int32, (_CHUNK, 1), 0)
                + i * _TILE
                + c * _CHUNK
            )
            update(jnp.where(row < n, chunk_scores(c), -jnp.inf))
        out_ref[...] = m_ref[...] + jnp.log(s_ref[...])


def _combine_body(q_ref, g_ref, logz_ref, out_ref):
    b = q_ref.shape[0]
    loss = (jnp.sum(logz_ref[...]) - jnp.sum(q_ref[...] * g_ref[...])) / b
    out_ref[...] = jnp.full((1, 1), loss, out_ref.dtype)


def kernel(query_embs, ent_embs, triplets):
    b, h = query_embs.shape
    n = ent_embs.shape[0]
    labels = triplets[:, 2].astype(jnp.int32)

    g = _make_sc_gather(b, h)(ent_embs, labels)

    grid = pl.cdiv(n, _TILE)
    logz = pl.pallas_call(
        functools.partial(_logz_body, n=n),
        grid=(grid,),
        in_specs=[
            pl.BlockSpec((h, b), lambda i: (0, 0)),
            pl.BlockSpec((_TILE, h), lambda i: (i, 0)),
        ],
        out_specs=pl.BlockSpec((1, b), lambda i: (0, 0)),
        out_shape=jax.ShapeDtypeStruct((1, b), jnp.float32),
        scratch_shapes=[
            pltpu.VMEM((1, b), jnp.float32),
            pltpu.VMEM((1, b), jnp.float32),
        ],
    )(query_embs.T.astype(jnp.bfloat16), ent_embs)

    loss = pl.pallas_call(
        _combine_body,
        out_shape=jax.ShapeDtypeStruct((1, 1), jnp.float32),
    )(query_embs, g, logz)
    return loss[0, 0]
